# baseline (speedup 1.0000x reference)
import functools

import jax
import jax.numpy as jnp
from jax import lax
from jax.experimental import pallas as pl
from jax.experimental.pallas import tpu as pltpu
from jax.experimental.pallas import tpu_sc as plsc

B = 16384
BB = 4096
NB = B // BB

NC, NS = 2, 16
NW = NC * NS
RPT = B // NW
CH = 128
NCH = RPT // CH

_EMB_SIZES = (20, 18, 16, 21)


def _sc_gather_body(c1_ref, c2_ref, c3_ref, c4_ref,
                    e1_ref, e2_ref, e3_ref, e4_ref,
                    o1_ref, o2_ref, o3_ref, o4_ref, idx_v, stage_v, sem):
    outs = (o1_ref, o2_ref, o3_ref, o4_ref)
    wid = lax.axis_index("s") * NC + lax.axis_index("c")
    row0 = wid * (RPT // CH)
    cats = (c1_ref, c2_ref, c3_ref, c4_ref)
    embs = (e1_ref, e2_ref, e3_ref, e4_ref)
    for s in range(4):
        pltpu.sync_copy(cats[s].at[pl.ds(row0, NCH)], idx_v)
        for c in range(NCH):
            pltpu.async_copy(
                embs[s].at[idx_v.at[c]],
                stage_v.at[pl.ds(c * CH, CH)],
                sem).wait()
        pltpu.sync_copy(stage_v, outs[s].at[pl.ds(wid * RPT, RPT)])


def _tc_mlp(x_ref, g1_ref, g2_ref, g3_ref, g4_ref,
            w1_ref, b1_ref, w2_ref, b2_ref, w3_ref, b3_ref, o_ref):
    f32 = jnp.float32
    acc = jnp.dot(x_ref[...], w1_ref[0:13, :], preferred_element_type=f32)
    zpad = jnp.zeros((96, 128), dtype=f32)
    for g_ref, w_lo in ((g1_ref, 13), (g2_ref, 45), (g3_ref, 77), (g4_ref, 109)):
        w_pad = jnp.concatenate([w1_ref[w_lo:w_lo + 32, :], zpad], axis=0)
        acc = acc + jnp.dot(g_ref[...], w_pad, preferred_element_type=f32)
    acc = acc + b1_ref[...]
    h1 = jnp.maximum(acc, 0.0)
    h2 = jnp.maximum(
        jnp.dot(h1, w2_ref[...], preferred_element_type=f32) + b2_ref[...], 0.0)
    o_ref[...] = jnp.dot(h2, w3_ref[...], preferred_element_type=f32) + b3_ref[...]


@functools.partial(jax.jit, static_argnums=())
def kernel(x, cat_1, cat_2, cat_3, occupation,
           emb1, emb2, emb3, emb_occ, W1, b1, W2, b2, W3, b3):
    mesh = plsc.VectorSubcoreMesh(core_axis_name="c", subcore_axis_name="s")
    sc_gather = functools.partial(
        pl.kernel, mesh=mesh,
        compiler_params=pltpu.CompilerParams(use_tc_tiling_on_sc=True),
        out_type=[jax.ShapeDtypeStruct((B, 128), jnp.float32)] * 4,
        scratch_types=[
            pltpu.VMEM((NCH, CH), jnp.int32),
            pltpu.VMEM((RPT, 128), jnp.float32),
            pltpu.SemaphoreType.DMA,
        ],
    )(_sc_gather_body)
    g1, g2, g3, g4 = sc_gather(
        cat_1.astype(jnp.int32).reshape(B // CH, CH),
        cat_2.astype(jnp.int32).reshape(B // CH, CH),
        cat_3.astype(jnp.int32).reshape(B // CH, CH),
        occupation.astype(jnp.int32).reshape(B // CH, CH),
        jnp.pad(emb1, ((0, 0), (0, 96))), jnp.pad(emb2, ((0, 0), (0, 96))),
        jnp.pad(emb3, ((0, 0), (0, 96))), jnp.pad(emb_occ, ((0, 0), (0, 96))))

    b1r = b1.reshape(1, 128)
    b2r = b2.reshape(1, 64)
    b3r = b3.reshape(1, 1)
    out = pl.pallas_call(
        _tc_mlp,
        grid=(NB,),
        in_specs=[
            pl.BlockSpec((BB, 13), lambda i: (i, 0)),
            pl.BlockSpec((BB, 128), lambda i: (i, 0)),
            pl.BlockSpec((BB, 128), lambda i: (i, 0)),
            pl.BlockSpec((BB, 128), lambda i: (i, 0)),
            pl.BlockSpec((BB, 128), lambda i: (i, 0)),
            pl.BlockSpec((141, 128), lambda i: (0, 0)),
            pl.BlockSpec((1, 128), lambda i: (0, 0)),
            pl.BlockSpec((128, 64), lambda i: (0, 0)),
            pl.BlockSpec((1, 64), lambda i: (0, 0)),
            pl.BlockSpec((64, 1), lambda i: (0, 0)),
            pl.BlockSpec((1, 1), lambda i: (0, 0)),
        ],
        out_specs=pl.BlockSpec((BB, 1), lambda i: (i, 0)),
        out_shape=jax.ShapeDtypeStruct((B, 1), jnp.float32),
        compiler_params=pltpu.CompilerParams(
            dimension_semantics=("arbitrary",)),
    )(x, g1, g2, g3, g4, W1, b1r, W2, b2r, W3, b3r)
    return out


# speedup vs baseline: 6.6282x; 6.6282x over previous
import functools

import jax
import jax.numpy as jnp
from jax.experimental import pallas as pl
from jax.experimental.pallas import tpu as pltpu

B = 16384
BB = 4096
NB = B // BB

_SLOTS = ((13, 20, 0), (45, 18, 20), (77, 16, 38), (109, 21, 54))
_TOT = 75


def _fwd_kernel(x_ref, idx_ref, e1_ref, e2_ref, e3_ref, e4_ref,
                w1_ref, b1_ref, w2_ref, b2_ref, w3_ref, b3_ref, o_ref):
    f32 = jnp.float32
    col = jax.lax.broadcasted_iota(jnp.int32, (4, 128), 1)
    row = jax.lax.broadcasted_iota(jnp.int32, (4, 128), 0)
    s_mat = jnp.zeros((4, 128), dtype=f32)
    iota_adj = jnp.full((1, 128), -1, dtype=f32)
    col1 = col[0:1, :]
    for s, (_, k, c_lo) in enumerate(_SLOTS):
        in_range = (col >= c_lo) & (col < c_lo + k)
        s_mat = jnp.where((row == s) & in_range, 1.0, s_mat)
        iota_adj = jnp.where((col1 >= c_lo) & (col1 < c_lo + k),
                             (col1 - c_lo).astype(f32), iota_adj)

    p = jnp.dot(idx_ref[...], s_mat, preferred_element_type=f32)
    m = (p == iota_adj).astype(f32)

    embs = (e1_ref, e2_ref, e3_ref, e4_ref)
    t_parts = [
        jnp.dot(embs[s][...], w1_ref[w_lo:w_lo + 32, :],
                preferred_element_type=f32)
        for s, (w_lo, k, _) in enumerate(_SLOTS)
    ]
    t_parts.append(jnp.zeros((128 - _TOT, 128), dtype=f32))
    t = jnp.concatenate(t_parts, axis=0)

    acc = jnp.dot(x_ref[...], w1_ref[0:13, :], preferred_element_type=f32)
    acc = acc + jnp.dot(m, t, preferred_element_type=f32) + b1_ref[...]
    h1 = jnp.maximum(acc, 0.0)
    h2 = jnp.maximum(
        jnp.dot(h1, w2_ref[...], preferred_element_type=f32) + b2_ref[...], 0.0)
    o_ref[...] = jnp.dot(h2, w3_ref[...], preferred_element_type=f32) + b3_ref[...]


@functools.partial(jax.jit, static_argnums=())
def kernel(x, cat_1, cat_2, cat_3, occupation,
           emb1, emb2, emb3, emb_occ, W1, b1, W2, b2, W3, b3):
    idx = jnp.stack([cat_1, cat_2, cat_3, occupation],
                    axis=1).astype(jnp.float32)
    b1r = b1.reshape(1, 128)
    b2r = b2.reshape(1, 64)
    b3r = b3.reshape(1, 1)

    out = pl.pallas_call(
        _fwd_kernel,
        grid=(NB,),
        in_specs=[
            pl.BlockSpec((BB, 13), lambda i: (i, 0)),
            pl.BlockSpec((BB, 4), lambda i: (i, 0)),
            pl.BlockSpec((20, 32), lambda i: (0, 0)),
            pl.BlockSpec((18, 32), lambda i: (0, 0)),
            pl.BlockSpec((16, 32), lambda i: (0, 0)),
            pl.BlockSpec((21, 32), lambda i: (0, 0)),
            pl.BlockSpec((141, 128), lambda i: (0, 0)),
            pl.BlockSpec((1, 128), lambda i: (0, 0)),
            pl.BlockSpec((128, 64), lambda i: (0, 0)),
            pl.BlockSpec((1, 64), lambda i: (0, 0)),
            pl.BlockSpec((64, 1), lambda i: (0, 0)),
            pl.BlockSpec((1, 1), lambda i: (0, 0)),
        ],
        out_specs=pl.BlockSpec((BB, 1), lambda i: (i, 0)),
        out_shape=jax.ShapeDtypeStruct((B, 1), jnp.float32),
        compiler_params=pltpu.CompilerParams(
            dimension_semantics=("arbitrary",)),
    )(x, idx, emb1, emb2, emb3, emb_occ, W1, b1r, W2, b2r, W3, b3r)
    return out
